# Initial kernel scaffold; baseline (speedup 1.0000x reference)
import jax
import jax.numpy as jnp
from jax.experimental import pallas as pl


def kernel(x_mirna, x_gene, seqG, Wm, bm, Wg, bg, basis1, comp1, root1, bias1, basis2, comp2, root2, bias2, W_ih_f, W_hh_f, b_ih_f, b_hh_f, W_ih_r, W_hh_r, b_ih_r, b_hh_r, W_lin, b_lin, W_embg, edge_index, edge_type, label_edge):
    raise NotImplementedError("write your pallas kernel here")


# speedup vs baseline: 3.7043x; 3.7043x over previous
import functools

import jax
import jax.numpy as jnp
from jax import lax
from jax.experimental import pallas as pl
from jax.experimental.pallas import tpu as pltpu
from jax.experimental.pallas import tpu_sc as plsc

N_MIRNA = 2000
N_GENE = 8000
N_NODES = N_MIRNA + N_GENE
NUM_REL = 4
E_TOT = 320000
T_SEQ = 64
B_EDGE = 16384

NCORE = 2
NSUB = 16
NW = NCORE * NSUB
EPT = E_TOT // NW
CHUNK = 80
NCHUNK = EPT // CHUNK
FS = 32
R4N = NUM_REL * N_NODES
NZW = 10
RPS = R4N // NZW
GK = 128
BB = 1000
NBLK = N_GENE // BB

_f32 = jnp.float32


def _sc_mesh():
    return plsc.VectorSubcoreMesh(core_axis_name="c", subcore_axis_name="s")


def _make_sc_agg(with_count):
    outs = [jax.ShapeDtypeStruct((4, NCORE, R4N, FS), _f32)]
    scr = [
        pltpu.VMEM((CHUNK,), jnp.int32),
        pltpu.VMEM((CHUNK,), jnp.int32),
        pltpu.VMEM((CHUNK, FS), _f32),
        pltpu.VMEM_SHARED((R4N, FS), _f32),
        pltpu.SemaphoreType.DMA,
    ]
    if with_count:
        outs.append(jax.ShapeDtypeStruct((NCORE, R4N, 16), _f32))
        scr += [
            pltpu.VMEM((CHUNK, 16), _f32),
            pltpu.VMEM_SHARED((R4N, 16), _f32),
        ]

    def body(t0, t1, t2, t3, gidx_hbm, sidx_hbm, z32_hbm, *rest):
        if with_count:
            (z16_hbm, o16_hbm, agg_out, cnt_out,
             gidx_v, sidx_v, rows_v, accS, sem, ones_v, accC) = rest
        else:
            (agg_out, gidx_v, sidx_v, rows_v, accS, sem) = rest
        c = lax.axis_index("c")
        s = lax.axis_index("s")
        wid = c * NSUB + s
        tabs = (t0, t1, t2, t3)
        if with_count:
            pltpu.sync_copy(o16_hbm, ones_v)
        for p in range(4):
            @pl.when(s < NZW)
            def _zero(p=p):
                pltpu.sync_copy(z32_hbm.at[pl.ds(s * RPS, RPS)],
                                accS.at[pl.ds(s * RPS, RPS)])
                if with_count and p == 0:
                    pltpu.sync_copy(z16_hbm.at[pl.ds(s * RPS, RPS)],
                                    accC.at[pl.ds(s * RPS, RPS)])
            plsc.subcore_barrier()

            def chunk(i, carry):
                b = wid * EPT + i * CHUNK
                pltpu.sync_copy(gidx_hbm.at[pl.ds(b, CHUNK)], gidx_v)
                pltpu.sync_copy(sidx_hbm.at[pl.ds(b, CHUNK)], sidx_v)
                pltpu.async_copy(tabs[p].at[gidx_v], rows_v, sem).wait()
                pltpu.sync_copy(rows_v, accS.at[sidx_v], add=True)
                if with_count and p == 0:
                    pltpu.sync_copy(ones_v, accC.at[sidx_v], add=True)
                return carry

            lax.fori_loop(0, NCHUNK, chunk, 0)
            plsc.subcore_barrier()

            @pl.when(s < NZW)
            def _copyout(p=p):
                pltpu.sync_copy(accS.at[pl.ds(s * RPS, RPS)],
                                agg_out.at[p, c, pl.ds(s * RPS, RPS)])
                if with_count and p == 0:
                    pltpu.sync_copy(accC.at[pl.ds(s * RPS, RPS)],
                                    cnt_out.at[c, pl.ds(s * RPS, RPS)])
            plsc.subcore_barrier()

    return pl.kernel(
        body, out_type=tuple(outs), mesh=_sc_mesh(), scratch_types=scr,
        compiler_params=pltpu.CompilerParams(use_tc_tiling_on_sc=False))


_sc_agg_count = _make_sc_agg(True)
_sc_agg = _make_sc_agg(False)


@functools.partial(
    pl.kernel,
    out_type=(jax.ShapeDtypeStruct((B_EDGE, 128), _f32),
              jax.ShapeDtypeStruct((B_EDGE, 128), _f32)),
    mesh=_sc_mesh(),
    scratch_types=[pltpu.VMEM((GK,), jnp.int32),
                   pltpu.VMEM((GK, 128), _f32),
                   pltpu.SemaphoreType.DMA],
)
def _sc_gather2(h2_hbm, xg_hbm, i0_hbm, i1_hbm, o0_hbm, o1_hbm,
                idx_v, rows_v, sem):
    c = lax.axis_index("c")
    s = lax.axis_index("s")
    wid = c * NSUB + s
    per = B_EDGE // NW

    def make_loop(table, ihbm, ohbm):
        def loop(i, carry):
            b = wid * per + i * GK
            pltpu.sync_copy(ihbm.at[pl.ds(b, GK)], idx_v)
            pltpu.async_copy(table.at[idx_v], rows_v, sem).wait()
            pltpu.sync_copy(rows_v, ohbm.at[pl.ds(b, GK)])
            return carry
        return loop

    lax.fori_loop(0, per // GK, make_loop(h2_hbm, i0_hbm, o0_hbm), 0)
    lax.fori_loop(0, per // GK, make_loop(xg_hbm, i1_hbm, o1_hbm), 0)


def _proj_body(xm_ref, wm_ref, bm_ref, xg_ref, wg_ref, bg_ref,
               r1_ref, b1_ref, x_ref, xr1_ref):
    xm = jnp.dot(xm_ref[...], wm_ref[...],
                 preferred_element_type=_f32) + bm_ref[...]
    xg = jnp.dot(xg_ref[...], wg_ref[...],
                 preferred_element_type=_f32) + bg_ref[...]
    x = jnp.concatenate([xm, xg], axis=0)
    x_ref[...] = x
    xr1_ref[...] = jnp.dot(x, r1_ref[...],
                           preferred_element_type=_f32) + b1_ref[...]


_proj = pl.pallas_call(
    _proj_body,
    out_shape=(jax.ShapeDtypeStruct((N_NODES, 128), _f32),
               jax.ShapeDtypeStruct((N_NODES, 128), _f32)),
)

NB = 1000
NGRID = N_NODES // NB


def _make_combine(with_root):
    def body(xr_ref, agg_ref, cnt_ref, comp_ref, basf_ref, *rest):
        if with_root:
            root_ref, b2_ref, h_ref, xr2_ref = rest
        else:
            (h_ref,) = rest
        wflat = jnp.dot(comp_ref[...], basf_ref[...],
                        preferred_element_type=_f32)
        wbig = jnp.reshape(wflat, (NUM_REL * 128, 128))
        cnt3 = jnp.sum(cnt_ref[0] + cnt_ref[1], axis=-1,
                       keepdims=True) * (1.0 / 16.0)
        recip = 1.0 / jnp.maximum(cnt3, 1.0)
        parts = []
        for r in range(NUM_REL):
            a_r = jnp.concatenate(
                [agg_ref[p, 0, r] + agg_ref[p, 1, r] for p in range(4)],
                axis=1)
            parts.append(a_r * recip[r])
        acat = jnp.concatenate(parts, axis=1)
        h = xr_ref[...] + jnp.dot(acat, wbig, preferred_element_type=_f32)
        h_ref[...] = h
        if with_root:
            xr2_ref[...] = jnp.dot(h, root_ref[...],
                                   preferred_element_type=_f32) + b2_ref[...]

    in_specs = [
        pl.BlockSpec((NB, 128), lambda i: (i, 0)),
        pl.BlockSpec((4, NCORE, NUM_REL, NB, FS), lambda i: (0, 0, 0, i, 0)),
        pl.BlockSpec((NCORE, NUM_REL, NB, 16), lambda i: (0, 0, i, 0)),
        pl.BlockSpec((NUM_REL, 10), lambda i: (0, 0)),
        pl.BlockSpec((10, 128 * 128), lambda i: (0, 0)),
    ]
    out_shape = [jax.ShapeDtypeStruct((N_NODES, 128), _f32)]
    if with_root:
        in_specs += [pl.BlockSpec((128, 128), lambda i: (0, 0)),
                     pl.BlockSpec((1, 128), lambda i: (0, 0))]
        out_shape.append(jax.ShapeDtypeStruct((N_NODES, 128), _f32))
    out_specs = [pl.BlockSpec((NB, 128), lambda i: (i, 0))] * len(out_shape)
    return pl.pallas_call(
        body, grid=(NGRID,), in_specs=in_specs,
        out_specs=out_specs if with_root else out_specs[0],
        out_shape=tuple(out_shape) if with_root else out_shape[0])


_combine1 = _make_combine(True)
_combine2 = _make_combine(False)


def _gates(xt, h, wih_ref, whh_ref, b_ref):
    g = (lax.dot_general(xt, wih_ref[...], (((1,), (1,)), ((), ())),
                         preferred_element_type=_f32)
         + lax.dot_general(h, whh_ref[...], (((1,), (1,)), ((), ())),
                           preferred_element_type=_f32)
         + b_ref[...])
    i_g = jax.nn.sigmoid(g[:, 0:32])
    f_g = jax.nn.sigmoid(g[:, 32:64])
    gg = jnp.tanh(g[:, 64:96])
    o_g = jax.nn.sigmoid(g[:, 96:128])
    return i_g, f_g, gg, o_g


def _lstm_f_body(seq_ref, wih_ref, whh_ref, b_ref, wl_ref, out_ref,
                 h_ref, c_ref):
    t = pl.program_id(1)

    @pl.when(t == 0)
    def _():
        h_ref[...] = jnp.zeros_like(h_ref)
        c_ref[...] = jnp.zeros_like(c_ref)
        out_ref[...] = jnp.zeros_like(out_ref)

    xt = seq_ref[0]
    i_g, f_g, gg, o_g = _gates(xt, h_ref[...], wih_ref, whh_ref, b_ref)
    c = f_g * c_ref[...] + i_g * gg
    h = o_g * jnp.tanh(c)
    c_ref[...] = c
    h_ref[...] = h
    out_ref[...] += jnp.dot(h, wl_ref[0], preferred_element_type=_f32)


_lstm_f = pl.pallas_call(
    _lstm_f_body,
    grid=(NBLK, T_SEQ),
    in_specs=[
        pl.BlockSpec((1, BB, 32), lambda i, t: (t, i, 0)),
        pl.BlockSpec((128, 32), lambda i, t: (0, 0)),
        pl.BlockSpec((128, 32), lambda i, t: (0, 0)),
        pl.BlockSpec((1, 128), lambda i, t: (0, 0)),
        pl.BlockSpec((1, 32, 128), lambda i, t: (t, 0, 0)),
    ],
    out_specs=pl.BlockSpec((BB, 128), lambda i, t: (i, 0)),
    out_shape=jax.ShapeDtypeStruct((N_GENE, 128), _f32),
    scratch_shapes=[pltpu.VMEM((BB, 32), _f32), pltpu.VMEM((BB, 32), _f32)],
)


def _lstm_r_body(seq_ref, wih_ref, whh_ref, b_ref, wl_ref, accf_ref,
                 xg1_ref, blin_ref, wemb_ref, out_ref, h_ref, c_ref):
    t = pl.program_id(1)

    @pl.when(t == 0)
    def _():
        h_ref[...] = jnp.zeros_like(h_ref)
        c_ref[...] = jnp.zeros_like(c_ref)
        out_ref[...] = jnp.zeros_like(out_ref)

    xt = seq_ref[0]
    i_g, f_g, gg, o_g = _gates(xt, h_ref[...], wih_ref, whh_ref, b_ref)
    c = f_g * c_ref[...] + i_g * gg
    h = o_g * jnp.tanh(c)
    c_ref[...] = c
    h_ref[...] = h
    out_ref[...] += jnp.dot(h, wl_ref[0], preferred_element_type=_f32)

    @pl.when(t == T_SEQ - 1)
    def _():
        xg2 = out_ref[...] + accf_ref[...] + blin_ref[...]
        out_ref[...] = (
            jnp.dot(xg1_ref[...], wemb_ref[0:128, :],
                    preferred_element_type=_f32)
            + jnp.dot(xg2, wemb_ref[128:256, :], preferred_element_type=_f32))


_lstm_r = pl.pallas_call(
    _lstm_r_body,
    grid=(NBLK, T_SEQ),
    in_specs=[
        pl.BlockSpec((1, BB, 32), lambda i, t: (T_SEQ - 1 - t, i, 0)),
        pl.BlockSpec((128, 32), lambda i, t: (0, 0)),
        pl.BlockSpec((128, 32), lambda i, t: (0, 0)),
        pl.BlockSpec((1, 128), lambda i, t: (0, 0)),
        pl.BlockSpec((1, 32, 128), lambda i, t: (T_SEQ - 1 - t, 0, 0)),
        pl.BlockSpec((BB, 128), lambda i, t: (i, 0)),
        pl.BlockSpec((BB, 128), lambda i, t: (i, 0)),
        pl.BlockSpec((1, 128), lambda i, t: (0, 0)),
        pl.BlockSpec((256, 128), lambda i, t: (0, 0)),
    ],
    out_specs=pl.BlockSpec((BB, 128), lambda i, t: (i, 0)),
    out_shape=jax.ShapeDtypeStruct((N_GENE, 128), _f32),
    scratch_shapes=[pltpu.VMEM((BB, 32), _f32), pltpu.VMEM((BB, 32), _f32)],
)


def _dots_body(r0_ref, r1_ref, out_ref):
    out_ref[...] = jnp.sum(r0_ref[...] * r1_ref[...], axis=-1)


_dots = pl.pallas_call(
    _dots_body,
    grid=(8,),
    in_specs=[pl.BlockSpec((16, 128, 128), lambda i: (i, 0, 0)),
              pl.BlockSpec((16, 128, 128), lambda i: (i, 0, 0))],
    out_specs=pl.BlockSpec((16, 128), lambda i: (i, 0)),
    out_shape=jax.ShapeDtypeStruct((128, 128), _f32),
)


def kernel(x_mirna, x_gene, seqG, Wm, bm, Wg, bg, basis1, comp1, root1,
           bias1, basis2, comp2, root2, bias2, W_ih_f, W_hh_f, b_ih_f,
           b_hh_f, W_ih_r, W_hh_r, b_ih_r, b_hh_r, W_lin, b_lin, W_embg,
           edge_index, edge_type, label_edge):
    src = edge_index[0]
    dst = edge_index[1]
    gidx = src.astype(jnp.int32)
    sidx = (edge_type * N_NODES + dst).astype(jnp.int32)

    x, xr1 = _proj(x_mirna, Wm, bm.reshape(1, 128), x_gene, Wg,
                   bg.reshape(1, 128), root1, bias1.reshape(1, 128))

    z32 = jnp.zeros((R4N, FS), _f32)
    z16 = jnp.zeros((R4N, 16), _f32)
    o16 = jnp.ones((CHUNK, 16), _f32)

    xs = x.reshape(N_NODES, 4, FS).transpose(1, 0, 2)
    agg1, cnt = _sc_agg_count(xs[0], xs[1], xs[2], xs[3], gidx, sidx,
                              z32, z16, o16)
    cntr = cnt.reshape(NCORE, NUM_REL, N_NODES, 16)

    h, xr2 = _combine1(xr1, agg1.reshape(4, NCORE, NUM_REL, N_NODES, FS),
                       cntr, comp1, basis1.reshape(10, 128 * 128),
                       root2, bias2.reshape(1, 128))

    hs = h.reshape(N_NODES, 4, FS).transpose(1, 0, 2)
    (agg2,) = _sc_agg(hs[0], hs[1], hs[2], hs[3], gidx, sidx, z32)

    h2 = _combine2(xr2, agg2.reshape(4, NCORE, NUM_REL, N_NODES, FS),
                   cntr, comp2, basis2.reshape(10, 128 * 128))

    wlin3 = W_lin.reshape(T_SEQ, 64, 128)
    xg1 = h2[N_MIRNA:]
    seqT = seqG.transpose(1, 0, 2)
    accf = _lstm_f(seqT, W_ih_f, W_hh_f, (b_ih_f + b_hh_f).reshape(1, 128),
                   wlin3[:, :32, :])
    xg = _lstm_r(seqT, W_ih_r, W_hh_r, (b_ih_r + b_hh_r).reshape(1, 128),
                 wlin3[:, 32:, :], accf, xg1, b_lin.reshape(1, 128), W_embg)

    r0, r1 = _sc_gather2(h2, xg, label_edge[0].astype(jnp.int32),
                         label_edge[1].astype(jnp.int32))
    s = _dots(r0.reshape(128, 128, 128), r1.reshape(128, 128, 128))
    return s.reshape(B_EDGE)


# speedup vs baseline: 4.6536x; 1.2563x over previous
import functools

import jax
import jax.numpy as jnp
from jax import lax
from jax.experimental import pallas as pl
from jax.experimental.pallas import tpu as pltpu
from jax.experimental.pallas import tpu_sc as plsc

N_MIRNA = 2000
N_GENE = 8000
N_NODES = N_MIRNA + N_GENE
NUM_REL = 4
E_TOT = 320000
T_SEQ = 64
B_EDGE = 16384

NCORE = 2
NSUB = 16
NW = NCORE * NSUB
CHUNK = 128
NCHUNK = 80
NBUF = 4
EPT = NCHUNK * CHUNK
EPAD = NW * EPT
FS = 32
R4N = NUM_REL * N_NODES
ACCR = R4N + 64
NZW = 10
RPS = R4N // NZW
GK = 128
BB = 1000
NBLK = N_GENE // BB

_f32 = jnp.float32


def _sc_mesh():
    return plsc.VectorSubcoreMesh(core_axis_name="c", subcore_axis_name="s")


def _make_sc_agg(with_count):
    outs = [jax.ShapeDtypeStruct((4, NCORE, R4N, FS), _f32)]
    scr = [
        pltpu.VMEM((NCHUNK, CHUNK), jnp.int32),
        pltpu.VMEM((NCHUNK, CHUNK), jnp.int32),
        tuple(pltpu.VMEM((CHUNK, FS), _f32) for _ in range(NBUF)),
        tuple(pltpu.SemaphoreType.DMA for _ in range(NBUF)),
        pltpu.VMEM_SHARED((ACCR, FS), _f32),
    ]
    if with_count:
        outs.append(jax.ShapeDtypeStruct((NCORE, R4N, FS), _f32))
        scr.append(pltpu.VMEM((CHUNK, FS), _f32))

    def body(t0, t1, t2, t3, gidx_hbm, sidx_hbm, z32_hbm, *rest):
        if with_count:
            (o32_hbm, agg_out, cnt_out,
             gidx_t, sidx_t, rows, semg, accS, ones_v) = rest
        else:
            (agg_out, gidx_t, sidx_t, rows, semg, accS) = rest
        c = lax.axis_index("c")
        s = lax.axis_index("s")
        wid = c * NSUB + s
        tabs = (t0, t1, t2, t3)
        pltpu.sync_copy(gidx_hbm.at[wid], gidx_t)
        pltpu.sync_copy(sidx_hbm.at[wid], sidx_t)
        if with_count:
            pltpu.sync_copy(o32_hbm, ones_v)
        npass = 5 if with_count else 4
        for p in range(npass):
            counting = p == 4
            @pl.when(s < NZW)
            def _zero():
                pltpu.sync_copy(z32_hbm.at[pl.ds(s * RPS, RPS)],
                                accS.at[pl.ds(s * RPS, RPS)])
            plsc.subcore_barrier()

            if counting:
                def cgroup(i, carry):
                    pltpu.sync_copy(ones_v, accS.at[sidx_t.at[i]], add=True)
                    return carry
                lax.fori_loop(0, NCHUNK, cgroup, 0)
            else:
                def gath(i, b, p=p):
                    return pltpu.make_async_copy(tabs[p].at[gidx_t.at[i]],
                                                 rows[b], semg[b])

                for b in range(NBUF):
                    gath(b, b).start()

                def group(g, carry, p=p):
                    for b in range(NBUF):
                        i = g * NBUF + b
                        gath(i, b).wait()
                        pltpu.sync_copy(rows[b], accS.at[sidx_t.at[i]],
                                        add=True)

                        @pl.when(i + NBUF < NCHUNK)
                        def _prefetch(i=i, b=b):
                            gath(i + NBUF, b).start()
                    return carry

                lax.fori_loop(0, NCHUNK // NBUF, group, 0)
            plsc.subcore_barrier()

            @pl.when(s < NZW)
            def _copyout(p=p, counting=counting):
                dst = (cnt_out.at[c, pl.ds(s * RPS, RPS)] if counting
                       else agg_out.at[p, c, pl.ds(s * RPS, RPS)])
                pltpu.sync_copy(accS.at[pl.ds(s * RPS, RPS)], dst)
            plsc.subcore_barrier()

    return pl.kernel(
        body, out_type=tuple(outs), mesh=_sc_mesh(), scratch_types=scr,
        compiler_params=pltpu.CompilerParams(use_tc_tiling_on_sc=False))


_sc_agg_count = _make_sc_agg(True)
_sc_agg = _make_sc_agg(False)


@functools.partial(
    pl.kernel,
    out_type=(jax.ShapeDtypeStruct((B_EDGE, 128), _f32),
              jax.ShapeDtypeStruct((B_EDGE, 128), _f32)),
    mesh=_sc_mesh(),
    scratch_types=[pltpu.VMEM((GK,), jnp.int32),
                   pltpu.VMEM((GK, 128), _f32),
                   pltpu.SemaphoreType.DMA],
)
def _sc_gather2(h2_hbm, xg_hbm, i0_hbm, i1_hbm, o0_hbm, o1_hbm,
                idx_v, rows_v, sem):
    c = lax.axis_index("c")
    s = lax.axis_index("s")
    wid = c * NSUB + s
    per = B_EDGE // NW

    def make_loop(table, ihbm, ohbm):
        def loop(i, carry):
            b = wid * per + i * GK
            pltpu.sync_copy(ihbm.at[pl.ds(b, GK)], idx_v)
            pltpu.async_copy(table.at[idx_v], rows_v, sem).wait()
            pltpu.sync_copy(rows_v, ohbm.at[pl.ds(b, GK)])
            return carry
        return loop

    lax.fori_loop(0, per // GK, make_loop(h2_hbm, i0_hbm, o0_hbm), 0)
    lax.fori_loop(0, per // GK, make_loop(xg_hbm, i1_hbm, o1_hbm), 0)


def _proj_body(xm_ref, wm_ref, bm_ref, xg_ref, wg_ref, bg_ref,
               r1_ref, b1_ref, x_ref, xr1_ref):
    xm = jnp.dot(xm_ref[...], wm_ref[...],
                 preferred_element_type=_f32) + bm_ref[...]
    xg = jnp.dot(xg_ref[...], wg_ref[...],
                 preferred_element_type=_f32) + bg_ref[...]
    x = jnp.concatenate([xm, xg], axis=0)
    x_ref[...] = x
    xr1_ref[...] = jnp.dot(x, r1_ref[...],
                           preferred_element_type=_f32) + b1_ref[...]


_proj = pl.pallas_call(
    _proj_body,
    out_shape=(jax.ShapeDtypeStruct((N_NODES, 128), _f32),
               jax.ShapeDtypeStruct((N_NODES, 128), _f32)),
)

NB = 1000
NGRID = N_NODES // NB


def _make_combine(with_root):
    def body(xr_ref, agg_ref, cnt_ref, comp_ref, basf_ref, *rest):
        if with_root:
            root_ref, b2_ref, h_ref, xr2_ref = rest
        else:
            (h_ref,) = rest
        wflat = jnp.dot(comp_ref[...], basf_ref[...],
                        preferred_element_type=_f32)
        wbig = jnp.reshape(wflat, (NUM_REL * 128, 128))
        cnt3 = jnp.sum(cnt_ref[0] + cnt_ref[1], axis=-1,
                       keepdims=True) * (1.0 / FS)
        recip = 1.0 / jnp.maximum(cnt3, 1.0)
        parts = []
        for r in range(NUM_REL):
            a_r = jnp.concatenate(
                [agg_ref[p, 0, r] + agg_ref[p, 1, r] for p in range(4)],
                axis=1)
            parts.append(a_r * recip[r])
        acat = jnp.concatenate(parts, axis=1)
        h = xr_ref[...] + jnp.dot(acat, wbig, preferred_element_type=_f32)
        h_ref[...] = h
        if with_root:
            xr2_ref[...] = jnp.dot(h, root_ref[...],
                                   preferred_element_type=_f32) + b2_ref[...]

    in_specs = [
        pl.BlockSpec((NB, 128), lambda i: (i, 0)),
        pl.BlockSpec((4, NCORE, NUM_REL, NB, FS), lambda i: (0, 0, 0, i, 0)),
        pl.BlockSpec((NCORE, NUM_REL, NB, FS), lambda i: (0, 0, i, 0)),
        pl.BlockSpec((NUM_REL, 10), lambda i: (0, 0)),
        pl.BlockSpec((10, 128 * 128), lambda i: (0, 0)),
    ]
    out_shape = [jax.ShapeDtypeStruct((N_NODES, 128), _f32)]
    if with_root:
        in_specs += [pl.BlockSpec((128, 128), lambda i: (0, 0)),
                     pl.BlockSpec((1, 128), lambda i: (0, 0))]
        out_shape.append(jax.ShapeDtypeStruct((N_NODES, 128), _f32))
    out_specs = [pl.BlockSpec((NB, 128), lambda i: (i, 0))] * len(out_shape)
    return pl.pallas_call(
        body, grid=(NGRID,), in_specs=in_specs,
        out_specs=out_specs if with_root else out_specs[0],
        out_shape=tuple(out_shape) if with_root else out_shape[0])


_combine1 = _make_combine(True)
_combine2 = _make_combine(False)


def _gates(xt, h, wih_ref, whh_ref, b_ref):
    g = (lax.dot_general(xt, wih_ref[...], (((1,), (1,)), ((), ())),
                         preferred_element_type=_f32)
         + lax.dot_general(h, whh_ref[...], (((1,), (1,)), ((), ())),
                           preferred_element_type=_f32)
         + b_ref[...])
    i_g = jax.nn.sigmoid(g[:, 0:32])
    f_g = jax.nn.sigmoid(g[:, 32:64])
    gg = jnp.tanh(g[:, 64:96])
    o_g = jax.nn.sigmoid(g[:, 96:128])
    return i_g, f_g, gg, o_g


def _lstm_f_body(seq_ref, wih_ref, whh_ref, b_ref, wl_ref, out_ref,
                 h_ref, c_ref):
    t = pl.program_id(1)

    @pl.when(t == 0)
    def _():
        h_ref[...] = jnp.zeros_like(h_ref)
        c_ref[...] = jnp.zeros_like(c_ref)
        out_ref[...] = jnp.zeros_like(out_ref)

    xt = seq_ref[0]
    i_g, f_g, gg, o_g = _gates(xt, h_ref[...], wih_ref, whh_ref, b_ref)
    c = f_g * c_ref[...] + i_g * gg
    h = o_g * jnp.tanh(c)
    c_ref[...] = c
    h_ref[...] = h
    out_ref[...] += jnp.dot(h, wl_ref[0], preferred_element_type=_f32)


_lstm_f = pl.pallas_call(
    _lstm_f_body,
    grid=(NBLK, T_SEQ),
    in_specs=[
        pl.BlockSpec((1, BB, 32), lambda i, t: (t, i, 0)),
        pl.BlockSpec((128, 32), lambda i, t: (0, 0)),
        pl.BlockSpec((128, 32), lambda i, t: (0, 0)),
        pl.BlockSpec((1, 128), lambda i, t: (0, 0)),
        pl.BlockSpec((1, 32, 128), lambda i, t: (t, 0, 0)),
    ],
    out_specs=pl.BlockSpec((BB, 128), lambda i, t: (i, 0)),
    out_shape=jax.ShapeDtypeStruct((N_GENE, 128), _f32),
    scratch_shapes=[pltpu.VMEM((BB, 32), _f32), pltpu.VMEM((BB, 32), _f32)],
)


def _lstm_r_body(seq_ref, wih_ref, whh_ref, b_ref, wl_ref, accf_ref,
                 xg1_ref, blin_ref, wemb_ref, out_ref, h_ref, c_ref):
    t = pl.program_id(1)

    @pl.when(t == 0)
    def _():
        h_ref[...] = jnp.zeros_like(h_ref)
        c_ref[...] = jnp.zeros_like(c_ref)
        out_ref[...] = jnp.zeros_like(out_ref)

    xt = seq_ref[0]
    i_g, f_g, gg, o_g = _gates(xt, h_ref[...], wih_ref, whh_ref, b_ref)
    c = f_g * c_ref[...] + i_g * gg
    h = o_g * jnp.tanh(c)
    c_ref[...] = c
    h_ref[...] = h
    out_ref[...] += jnp.dot(h, wl_ref[0], preferred_element_type=_f32)

    @pl.when(t == T_SEQ - 1)
    def _():
        xg2 = out_ref[...] + accf_ref[...] + blin_ref[...]
        out_ref[...] = (
            jnp.dot(xg1_ref[...], wemb_ref[0:128, :],
                    preferred_element_type=_f32)
            + jnp.dot(xg2, wemb_ref[128:256, :], preferred_element_type=_f32))


_lstm_r = pl.pallas_call(
    _lstm_r_body,
    grid=(NBLK, T_SEQ),
    in_specs=[
        pl.BlockSpec((1, BB, 32), lambda i, t: (T_SEQ - 1 - t, i, 0)),
        pl.BlockSpec((128, 32), lambda i, t: (0, 0)),
        pl.BlockSpec((128, 32), lambda i, t: (0, 0)),
        pl.BlockSpec((1, 128), lambda i, t: (0, 0)),
        pl.BlockSpec((1, 32, 128), lambda i, t: (T_SEQ - 1 - t, 0, 0)),
        pl.BlockSpec((BB, 128), lambda i, t: (i, 0)),
        pl.BlockSpec((BB, 128), lambda i, t: (i, 0)),
        pl.BlockSpec((1, 128), lambda i, t: (0, 0)),
        pl.BlockSpec((256, 128), lambda i, t: (0, 0)),
    ],
    out_specs=pl.BlockSpec((BB, 128), lambda i, t: (i, 0)),
    out_shape=jax.ShapeDtypeStruct((N_GENE, 128), _f32),
    scratch_shapes=[pltpu.VMEM((BB, 32), _f32), pltpu.VMEM((BB, 32), _f32)],
)


def _dots_body(r0_ref, r1_ref, out_ref):
    out_ref[...] = jnp.sum(r0_ref[...] * r1_ref[...], axis=-1)


_dots = pl.pallas_call(
    _dots_body,
    grid=(8,),
    in_specs=[pl.BlockSpec((16, 128, 128), lambda i: (i, 0, 0)),
              pl.BlockSpec((16, 128, 128), lambda i: (i, 0, 0))],
    out_specs=pl.BlockSpec((16, 128), lambda i: (i, 0)),
    out_shape=jax.ShapeDtypeStruct((128, 128), _f32),
)


def kernel(x_mirna, x_gene, seqG, Wm, bm, Wg, bg, basis1, comp1, root1,
           bias1, basis2, comp2, root2, bias2, W_ih_f, W_hh_f, b_ih_f,
           b_hh_f, W_ih_r, W_hh_r, b_ih_r, b_hh_r, W_lin, b_lin, W_embg,
           edge_index, edge_type, label_edge):
    src = edge_index[0]
    dst = edge_index[1]
    npad = EPAD - E_TOT
    gidx = jnp.concatenate(
        [src.astype(jnp.int32), jnp.zeros((npad,), jnp.int32)]
    ).reshape(NW, NCHUNK, CHUNK)
    sidx = jnp.concatenate(
        [(edge_type * N_NODES + dst).astype(jnp.int32),
         jnp.full((npad,), R4N, jnp.int32)]
    ).reshape(NW, NCHUNK, CHUNK)

    x, xr1 = _proj(x_mirna, Wm, bm.reshape(1, 128), x_gene, Wg,
                   bg.reshape(1, 128), root1, bias1.reshape(1, 128))

    z32 = jnp.zeros((R4N, FS), _f32)
    o32 = jnp.ones((CHUNK, FS), _f32)

    xs = x.reshape(N_NODES, 4, FS).transpose(1, 0, 2)
    agg1, cnt = _sc_agg_count(xs[0], xs[1], xs[2], xs[3], gidx, sidx,
                              z32, o32)
    cntr = cnt.reshape(NCORE, NUM_REL, N_NODES, FS)

    h, xr2 = _combine1(xr1, agg1.reshape(4, NCORE, NUM_REL, N_NODES, FS),
                       cntr, comp1, basis1.reshape(10, 128 * 128),
                       root2, bias2.reshape(1, 128))

    hs = h.reshape(N_NODES, 4, FS).transpose(1, 0, 2)
    (agg2,) = _sc_agg(hs[0], hs[1], hs[2], hs[3], gidx, sidx, z32)

    h2 = _combine2(xr2, agg2.reshape(4, NCORE, NUM_REL, N_NODES, FS),
                   cntr, comp2, basis2.reshape(10, 128 * 128))

    wlin3 = W_lin.reshape(T_SEQ, 64, 128)
    xg1 = h2[N_MIRNA:]
    seqT = seqG.transpose(1, 0, 2)
    accf = _lstm_f(seqT, W_ih_f, W_hh_f, (b_ih_f + b_hh_f).reshape(1, 128),
                   wlin3[:, :32, :])
    xg = _lstm_r(seqT, W_ih_r, W_hh_r, (b_ih_r + b_hh_r).reshape(1, 128),
                 wlin3[:, 32:, :], accf, xg1, b_lin.reshape(1, 128), W_embg)

    r0, r1 = _sc_gather2(h2, xg, label_edge[0].astype(jnp.int32),
                         label_edge[1].astype(jnp.int32))
    s = _dots(r0.reshape(128, 128, 128), r1.reshape(128, 128, 128))
    return s.reshape(B_EDGE)
